# baseline (speedup 1.0000x reference)
import functools

import jax
import jax.numpy as jnp
from jax import lax
from jax.experimental import pallas as pl
from jax.experimental.pallas import tpu as pltpu
from jax.experimental.pallas import tpu_sc as plsc

_N = 10000
_E = 320000
_D = 128
_NP = 10240
_DUMP = _NP - 1
_NW = 32
_E2 = 327680
_EW = _E2 // _NW
_CH = 128
_NC = _EW // _CH
_SB = 8
_NSBT = _E2 // (_SB * _CH)
_NSB0 = _NSBT // 16
_BM = 512
_NB = _NP // _BM
_RPS = _NP // 16

_mesh = plsc.VectorSubcoreMesh(core_axis_name="c", subcore_axis_name="s")


def _idx_prep_body(dst_ref, attr_ref, s1_ref, s2_ref):
    a = attr_ref[...]
    d = dst_ref[...]
    dump = _N + lax.broadcasted_iota(jnp.int32, a.shape, 1)
    s1_ref[...] = jnp.where(a == 1, d, dump)
    s2_ref[...] = jnp.where(a == 2, d, dump)


def _idx_prep(dst2d, attr2d):
    return pl.pallas_call(
        _idx_prep_body,
        out_shape=[jax.ShapeDtypeStruct((_E2 // 128, 128), jnp.int32)] * 2,
    )(dst2d, attr2d)


@functools.partial(
    pl.kernel,
    out_type=jax.ShapeDtypeStruct((2, 2 * _NP), jnp.float32),
    mesh=_mesh,
    scratch_types=[
        pltpu.VMEM((_NC, _CH), jnp.int32),
        pltpu.VMEM((_NC, _CH), jnp.int32),
        pltpu.VMEM((_CH,), jnp.float32),
        pltpu.VMEM((_RPS,), jnp.float32),
        pltpu.VMEM_SHARED((_NP,), jnp.float32),
        pltpu.VMEM_SHARED((_NP,), jnp.float32),
    ],
)
def _deg_kernel(s1_hbm, s2_hbm, ones_hbm, zeros_hbm, degp_hbm,
                s1_v, s2_v, ones_v, zb_v, cnt1_sh, cnt2_sh):
    c = lax.axis_index("c")
    s = lax.axis_index("s")
    w = s * 2 + c

    pltpu.sync_copy(ones_hbm, ones_v)
    pltpu.sync_copy(zeros_hbm, zb_v)
    pltpu.sync_copy(zb_v, cnt1_sh.at[pl.ds(s * _RPS, _RPS)])
    pltpu.sync_copy(zb_v, cnt2_sh.at[pl.ds(s * _RPS, _RPS)])

    plsc.subcore_barrier()

    pltpu.sync_copy(s1_hbm.at[w], s1_v)
    pltpu.sync_copy(s2_hbm.at[w], s2_v)

    @pl.loop(0, _NC)
    def _chunks(j):
        pltpu.sync_copy(ones_v, cnt1_sh.at[s1_v.at[j]], add=True)
        pltpu.sync_copy(ones_v, cnt2_sh.at[s2_v.at[j]], add=True)

    plsc.subcore_barrier()

    for k, cnt_sh in enumerate((cnt1_sh, cnt2_sh)):
        r = s * _RPS
        pltpu.sync_copy(cnt_sh.at[pl.ds(r, _RPS)], zb_v)
        pltpu.sync_copy(zb_v, degp_hbm.at[c, pl.ds(k * _NP + r, _RPS)])


def _dinv_body(degp_ref, dinv_ref):
    dinv_ref[...] = lax.rsqrt(degp_ref[0] + degp_ref[1] + 1.0)


def _dinv(degp):
    return pl.pallas_call(
        _dinv_body,
        out_shape=jax.ShapeDtypeStruct((2, _NP), jnp.float32),
    )(degp)


@functools.partial(
    pl.kernel,
    out_type=jax.ShapeDtypeStruct((_NP, _D), jnp.float32),
    mesh=_mesh,
    scratch_types=[
        pltpu.VMEM((_SB, _CH), jnp.int32),
        pltpu.VMEM((_SB, _CH), jnp.int32),
        pltpu.VMEM((2, _CH, _D), jnp.float32),
        pltpu.VMEM_SHARED((_NP, _D), jnp.float32),
        pltpu.SemaphoreType.DMA,
        pltpu.SemaphoreType.DMA,
    ],
)
def _agg_kernel(table_hbm, gidx_hbm, sidx_hbm, z_hbm, out_hbm,
                gidx_v, sidx_v, rows_v, acc_sh, sem0, sem1):
    c = lax.axis_index("c")
    s = lax.axis_index("s")
    sems = (sem0, sem1)

    @pl.when(c == 0)
    def _sc0_body():
        zsrc = rows_v.at[0, pl.ds(0, 64)]
        pltpu.sync_copy(z_hbm, zsrc)
        for k in range(_RPS // 64):
            pltpu.sync_copy(zsrc, acc_sh.at[pl.ds(s * _RPS + k * 64, 64)])

        plsc.subcore_barrier()

        @pl.loop(0, _NSB0)
        def _super(u):
            sb = s * _NSB0 + u
            pltpu.sync_copy(gidx_hbm.at[sb], gidx_v)
            pltpu.sync_copy(sidx_hbm.at[sb], sidx_v)

            pltpu.async_copy(table_hbm.at[gidx_v.at[0]], rows_v.at[0], sem0)
            pltpu.async_copy(table_hbm.at[gidx_v.at[1]], rows_v.at[1], sem1)

            @pl.loop(0, _SB, step=2)
            def _chunks(j):
                for b in range(2):
                    jj = j + b
                    pltpu.make_async_copy(
                        table_hbm.at[gidx_v.at[jj]], rows_v.at[b],
                        sems[b]).wait()
                    pltpu.sync_copy(rows_v.at[b], acc_sh.at[sidx_v.at[jj]],
                                    add=True)

                    @pl.when(jj + 2 < _SB)
                    def _():
                        pltpu.async_copy(
                            table_hbm.at[gidx_v.at[jj + 2]], rows_v.at[b],
                            sems[b])

        plsc.subcore_barrier()

        bounce = rows_v.at[0, pl.ds(0, 64)]
        for k in range(_RPS // 64):
            r = s * _RPS + k * 64
            pltpu.sync_copy(acc_sh.at[pl.ds(r, 64)], bounce)
            pltpu.sync_copy(bounce, out_hbm.at[pl.ds(r, 64)])


def _mm_scale_body(x_ref, w_ref, dinv_ref, out_ref):
    h = jnp.dot(x_ref[...], w_ref[...], preferred_element_type=jnp.float32)
    out_ref[...] = h * dinv_ref[...][:, None]


def _mm_scale(xp, W, dinv1):
    return pl.pallas_call(
        _mm_scale_body,
        grid=(_NB,),
        in_specs=[
            pl.BlockSpec((_BM, _D), lambda i: (i, 0)),
            pl.BlockSpec((_D, _D), lambda i: (0, 0)),
            pl.BlockSpec((_BM,), lambda i: (i,)),
        ],
        out_specs=pl.BlockSpec((_BM, _D), lambda i: (i, 0)),
        out_shape=jax.ShapeDtypeStruct((_NP, _D), jnp.float32),
    )(xp, W, dinv1)


def _mm2_body(x1_ref, x0_ref, w2_ref, w3_ref, dinv_ref, h2_ref, h3_ref):
    h2 = jnp.dot(x1_ref[...], w2_ref[...], preferred_element_type=jnp.float32)
    h3 = jnp.dot(x0_ref[...], w3_ref[...], preferred_element_type=jnp.float32)
    h2_ref[...] = h2 * dinv_ref[0][:, None]
    h3_ref[...] = h3 * dinv_ref[1][:, None]


def _mm2(x1, xp, W2, W3, dinv):
    return pl.pallas_call(
        _mm2_body,
        grid=(_NB,),
        in_specs=[
            pl.BlockSpec((_BM, _D), lambda i: (i, 0)),
            pl.BlockSpec((_BM, _D), lambda i: (i, 0)),
            pl.BlockSpec((_D, _D), lambda i: (0, 0)),
            pl.BlockSpec((_D, _D), lambda i: (0, 0)),
            pl.BlockSpec((2, _BM), lambda i: (0, i)),
        ],
        out_specs=[pl.BlockSpec((_BM, _D), lambda i: (i, 0))] * 2,
        out_shape=[jax.ShapeDtypeStruct((_NP, _D), jnp.float32)] * 2,
    )(x1, xp, W2, W3, dinv)


def _combine1_body(x_ref, hs_ref, aggp_ref, dinv_ref, b_ref, out_ref):
    agg = aggp_ref[...] + hs_ref[...]
    o = agg * dinv_ref[...][:, None] + b_ref[...][None, :]
    t = x_ref[...] + jnp.maximum(o, 0.0)
    nrm = jnp.sqrt(jnp.sum(t * t, axis=-1, keepdims=True))
    out_ref[...] = t / jnp.maximum(nrm, 1e-12)


def _combine1(xp, h1s, aggp, dinv1, b1):
    return pl.pallas_call(
        _combine1_body,
        grid=(_NB,),
        in_specs=[
            pl.BlockSpec((_BM, _D), lambda i: (i, 0)),
            pl.BlockSpec((_BM, _D), lambda i: (i, 0)),
            pl.BlockSpec((_BM, _D), lambda i: (i, 0)),
            pl.BlockSpec((_BM,), lambda i: (i,)),
            pl.BlockSpec((_D,), lambda i: (0,)),
        ],
        out_specs=pl.BlockSpec((_BM, _D), lambda i: (i, 0)),
        out_shape=jax.ShapeDtypeStruct((_NP, _D), jnp.float32),
    )(xp, h1s, aggp, dinv1, b1)


def _combine2_body(x1_ref, h2_ref, h3_ref, agg2_ref, agg3_ref, dinv_ref,
                   b2_ref, b3_ref, out_ref):
    a2 = (agg2_ref[...] + h2_ref[...]) * dinv_ref[0][:, None] \
        + b2_ref[...][None, :]
    a3 = (agg3_ref[...] + h3_ref[...]) * dinv_ref[1][:, None] \
        + b3_ref[...][None, :]
    t = x1_ref[...] + jnp.maximum(a2 + a3, 0.0)
    nrm = jnp.sqrt(jnp.sum(t * t, axis=-1, keepdims=True))
    out_ref[...] = t / jnp.maximum(nrm, 1e-12)


def _combine2(x1, h2s, h3s, agg2, agg3, dinv, b2, b3):
    return pl.pallas_call(
        _combine2_body,
        grid=(_NB,),
        in_specs=[
            pl.BlockSpec((_BM, _D), lambda i: (i, 0)),
            pl.BlockSpec((_BM, _D), lambda i: (i, 0)),
            pl.BlockSpec((_BM, _D), lambda i: (i, 0)),
            pl.BlockSpec((_BM, _D), lambda i: (i, 0)),
            pl.BlockSpec((_BM, _D), lambda i: (i, 0)),
            pl.BlockSpec((2, _BM), lambda i: (0, i)),
            pl.BlockSpec((_D,), lambda i: (0,)),
            pl.BlockSpec((_D,), lambda i: (0,)),
        ],
        out_specs=pl.BlockSpec((_BM, _D), lambda i: (i, 0)),
        out_shape=jax.ShapeDtypeStruct((_NP, _D), jnp.float32),
    )(x1, h2s, h3s, agg2, agg3, dinv, b2, b3)


def kernel(x, edge_index, edge_attr, W1, b1, W2, b2, W3, b3):
    src = edge_index[0]
    dst = edge_index[1]
    xp = jnp.pad(x, ((0, _NP - _N), (0, 0)))
    pad_e = _E2 - _E
    dstp = jnp.pad(dst, (0, pad_e))
    attrp = jnp.pad(edge_attr, (0, pad_e))
    srcp = jnp.pad(src, (0, pad_e))

    s1, s2 = _idx_prep(dstp.reshape(_E2 // 128, 128),
                       attrp.reshape(_E2 // 128, 128))
    s1 = s1.reshape(_E2)
    s2 = s2.reshape(_E2)
    s1_3d = s1.reshape(_NW, _NC, _CH)
    s2_3d = s2.reshape(_NW, _NC, _CH)
    s1_4d = s1.reshape(_NSBT, _SB, _CH)
    s2_4d = s2.reshape(_NSBT, _SB, _CH)
    g_4d = srcp.reshape(_NSBT, _SB, _CH)

    ones_c = jnp.ones((_CH,), jnp.float32)
    zeros_r = jnp.zeros((_RPS,), jnp.float32)
    zeros_b = jnp.zeros((64, _D), jnp.float32)

    degp = _deg_kernel(s1_3d, s2_3d, ones_c, zeros_r)
    dinv = _dinv(degp.reshape(2, 2, _NP))

    h1s = _mm_scale(xp, W1, dinv[0])
    agg1 = _agg_kernel(h1s, g_4d, s1_4d, zeros_b)
    x1 = _combine1(xp, h1s, agg1, dinv[0], b1)

    h2s, h3s = _mm2(x1, xp, W2, W3, dinv)
    agg2 = _agg_kernel(h2s, g_4d, s1_4d, zeros_b)
    agg3 = _agg_kernel(h3s, g_4d, s2_4d, zeros_b)
    x2 = _combine2(x1, h2s, h3s, agg2, agg3, dinv, b2, b3)

    return x2[:_N]


# speedup vs baseline: 3.1421x; 3.1421x over previous
import functools

import jax
import jax.numpy as jnp
from jax import lax
from jax.experimental import pallas as pl
from jax.experimental.pallas import tpu as pltpu
from jax.experimental.pallas import tpu_sc as plsc

_N = 10000
_E = 320000
_D = 128
_NP = 10240
_DUMP = _NP - 1
_NW = 32
_E2 = 327680
_EW = _E2 // _NW
_CH = 128
_NC = _EW // _CH
_SB = 8
_NSBT = _E2 // (_SB * _CH)
_NSBW = _NSBT // _NW
_BM = 512
_NB = _NP // _BM
_RPS = _NP // 16

_mesh = plsc.VectorSubcoreMesh(core_axis_name="c", subcore_axis_name="s")


def _idx_prep_body(dst_ref, attr_ref, s1_ref, s2_ref):
    a = attr_ref[...]
    d = dst_ref[...]
    dump = _N + lax.broadcasted_iota(jnp.int32, a.shape, 1)
    s1_ref[...] = jnp.where(a == 1, d, dump)
    s2_ref[...] = jnp.where(a == 2, d, dump)


def _idx_prep(dst2d, attr2d):
    return pl.pallas_call(
        _idx_prep_body,
        out_shape=[jax.ShapeDtypeStruct((_E2 // 128, 128), jnp.int32)] * 2,
    )(dst2d, attr2d)


@functools.partial(
    pl.kernel,
    out_type=jax.ShapeDtypeStruct((2, 2 * _NP), jnp.float32),
    mesh=_mesh,
    scratch_types=[
        pltpu.VMEM((_NC, _CH), jnp.int32),
        pltpu.VMEM((_NC, _CH), jnp.int32),
        pltpu.VMEM((_CH,), jnp.float32),
        pltpu.VMEM((_RPS,), jnp.float32),
        pltpu.VMEM_SHARED((_NP,), jnp.float32),
        pltpu.VMEM_SHARED((_NP,), jnp.float32),
    ],
)
def _deg_kernel(s1_hbm, s2_hbm, ones_hbm, zeros_hbm, degp_hbm,
                s1_v, s2_v, ones_v, zb_v, cnt1_sh, cnt2_sh):
    c = lax.axis_index("c")
    s = lax.axis_index("s")
    w = s * 2 + c

    pltpu.sync_copy(ones_hbm, ones_v)
    pltpu.sync_copy(zeros_hbm, zb_v)
    pltpu.sync_copy(zb_v, cnt1_sh.at[pl.ds(s * _RPS, _RPS)])
    pltpu.sync_copy(zb_v, cnt2_sh.at[pl.ds(s * _RPS, _RPS)])

    plsc.subcore_barrier()

    pltpu.sync_copy(s1_hbm.at[w], s1_v)
    pltpu.sync_copy(s2_hbm.at[w], s2_v)

    @pl.loop(0, _NC)
    def _chunks(j):
        pltpu.sync_copy(ones_v, cnt1_sh.at[s1_v.at[j]], add=True)
        pltpu.sync_copy(ones_v, cnt2_sh.at[s2_v.at[j]], add=True)

    plsc.subcore_barrier()

    for k, cnt_sh in enumerate((cnt1_sh, cnt2_sh)):
        r = s * _RPS
        pltpu.sync_copy(cnt_sh.at[pl.ds(r, _RPS)], zb_v)
        pltpu.sync_copy(zb_v, degp_hbm.at[c, pl.ds(k * _NP + r, _RPS)])


def _dinv_body(degp_ref, dinv_ref):
    dinv_ref[...] = lax.rsqrt(degp_ref[0] + degp_ref[1] + 1.0)


def _dinv(degp):
    return pl.pallas_call(
        _dinv_body,
        out_shape=jax.ShapeDtypeStruct((2, _NP), jnp.float32),
    )(degp)


@functools.partial(
    pl.kernel,
    out_type=jax.ShapeDtypeStruct((2, _NP, _D), jnp.float32),
    mesh=_mesh,
    scratch_types=[
        pltpu.VMEM((_SB, _CH), jnp.int32),
        pltpu.VMEM((_SB, _CH), jnp.int32),
        pltpu.VMEM((2, _CH, _D), jnp.float32),
        pltpu.VMEM_SHARED((_NP, _D), jnp.float32),
        pltpu.SemaphoreType.DMA,
        pltpu.SemaphoreType.DMA,
    ],
)
def _agg_kernel(table_hbm, gidx_hbm, sidx_hbm, z_hbm, out_hbm,
                gidx_v, sidx_v, rows_v, acc_sh, sem0, sem1):
    c = lax.axis_index("c")
    s = lax.axis_index("s")
    w = s * 2 + c
    sems = (sem0, sem1)

    zsrc = rows_v.at[0, pl.ds(0, 64)]
    pltpu.sync_copy(z_hbm, zsrc)
    for k in range(_RPS // 64):
        pltpu.sync_copy(zsrc, acc_sh.at[pl.ds(s * _RPS + k * 64, 64)])

    plsc.subcore_barrier()

    @pl.loop(0, _NSBW)
    def _super(u):
        sb = w * _NSBW + u
        pltpu.sync_copy(gidx_hbm.at[sb], gidx_v)
        pltpu.sync_copy(sidx_hbm.at[sb], sidx_v)

        pltpu.async_copy(table_hbm.at[gidx_v.at[0]], rows_v.at[0], sem0)
        pltpu.async_copy(table_hbm.at[gidx_v.at[1]], rows_v.at[1], sem1)

        @pl.loop(0, _SB, step=2)
        def _chunks(j):
            for b in range(2):
                jj = j + b
                pltpu.make_async_copy(
                    table_hbm.at[gidx_v.at[jj]], rows_v.at[b],
                    sems[b]).wait()
                pltpu.sync_copy(rows_v.at[b], acc_sh.at[sidx_v.at[jj]],
                                add=True)

                @pl.when(jj + 2 < _SB)
                def _():
                    pltpu.async_copy(
                        table_hbm.at[gidx_v.at[jj + 2]], rows_v.at[b],
                        sems[b])

    plsc.subcore_barrier()

    bounce = rows_v.at[0, pl.ds(0, 64)]
    for k in range(_RPS // 64):
        r = s * _RPS + k * 64
        pltpu.sync_copy(acc_sh.at[pl.ds(r, 64)], bounce)
        pltpu.sync_copy(bounce, out_hbm.at[c, pl.ds(r, 64)])


def _mm_scale_body(x_ref, w_ref, dinv_ref, out_ref):
    h = jnp.dot(x_ref[...], w_ref[...], preferred_element_type=jnp.float32)
    out_ref[...] = h * dinv_ref[...][:, None]


def _mm_scale(xp, W, dinv1):
    return pl.pallas_call(
        _mm_scale_body,
        grid=(_NB,),
        in_specs=[
            pl.BlockSpec((_BM, _D), lambda i: (i, 0)),
            pl.BlockSpec((_D, _D), lambda i: (0, 0)),
            pl.BlockSpec((_BM,), lambda i: (i,)),
        ],
        out_specs=pl.BlockSpec((_BM, _D), lambda i: (i, 0)),
        out_shape=jax.ShapeDtypeStruct((_NP, _D), jnp.float32),
    )(xp, W, dinv1)


def _mm2_body(x1_ref, x0_ref, w2_ref, w3_ref, dinv_ref, h2_ref, h3_ref):
    h2 = jnp.dot(x1_ref[...], w2_ref[...], preferred_element_type=jnp.float32)
    h3 = jnp.dot(x0_ref[...], w3_ref[...], preferred_element_type=jnp.float32)
    h2_ref[...] = h2 * dinv_ref[0][:, None]
    h3_ref[...] = h3 * dinv_ref[1][:, None]


def _mm2(x1, xp, W2, W3, dinv):
    return pl.pallas_call(
        _mm2_body,
        grid=(_NB,),
        in_specs=[
            pl.BlockSpec((_BM, _D), lambda i: (i, 0)),
            pl.BlockSpec((_BM, _D), lambda i: (i, 0)),
            pl.BlockSpec((_D, _D), lambda i: (0, 0)),
            pl.BlockSpec((_D, _D), lambda i: (0, 0)),
            pl.BlockSpec((2, _BM), lambda i: (0, i)),
        ],
        out_specs=[pl.BlockSpec((_BM, _D), lambda i: (i, 0))] * 2,
        out_shape=[jax.ShapeDtypeStruct((_NP, _D), jnp.float32)] * 2,
    )(x1, xp, W2, W3, dinv)


def _combine1_body(x_ref, hs_ref, aggp_ref, dinv_ref, b_ref, out_ref):
    agg = aggp_ref[0] + aggp_ref[1] + hs_ref[...]
    o = agg * dinv_ref[...][:, None] + b_ref[...][None, :]
    t = x_ref[...] + jnp.maximum(o, 0.0)
    nrm = jnp.sqrt(jnp.sum(t * t, axis=-1, keepdims=True))
    out_ref[...] = t / jnp.maximum(nrm, 1e-12)


def _combine1(xp, h1s, aggp, dinv1, b1):
    return pl.pallas_call(
        _combine1_body,
        grid=(_NB,),
        in_specs=[
            pl.BlockSpec((_BM, _D), lambda i: (i, 0)),
            pl.BlockSpec((_BM, _D), lambda i: (i, 0)),
            pl.BlockSpec((2, _BM, _D), lambda i: (0, i, 0)),
            pl.BlockSpec((_BM,), lambda i: (i,)),
            pl.BlockSpec((_D,), lambda i: (0,)),
        ],
        out_specs=pl.BlockSpec((_BM, _D), lambda i: (i, 0)),
        out_shape=jax.ShapeDtypeStruct((_NP, _D), jnp.float32),
    )(xp, h1s, aggp, dinv1, b1)


def _combine2_body(x1_ref, h2_ref, h3_ref, agg2_ref, agg3_ref, dinv_ref,
                   b2_ref, b3_ref, out_ref):
    a2 = (agg2_ref[0] + agg2_ref[1] + h2_ref[...]) * dinv_ref[0][:, None] \
        + b2_ref[...][None, :]
    a3 = (agg3_ref[0] + agg3_ref[1] + h3_ref[...]) * dinv_ref[1][:, None] \
        + b3_ref[...][None, :]
    t = x1_ref[...] + jnp.maximum(a2 + a3, 0.0)
    nrm = jnp.sqrt(jnp.sum(t * t, axis=-1, keepdims=True))
    out_ref[...] = t / jnp.maximum(nrm, 1e-12)


def _combine2(x1, h2s, h3s, agg2, agg3, dinv, b2, b3):
    return pl.pallas_call(
        _combine2_body,
        grid=(_NB,),
        in_specs=[
            pl.BlockSpec((_BM, _D), lambda i: (i, 0)),
            pl.BlockSpec((_BM, _D), lambda i: (i, 0)),
            pl.BlockSpec((_BM, _D), lambda i: (i, 0)),
            pl.BlockSpec((2, _BM, _D), lambda i: (0, i, 0)),
            pl.BlockSpec((2, _BM, _D), lambda i: (0, i, 0)),
            pl.BlockSpec((2, _BM), lambda i: (0, i)),
            pl.BlockSpec((_D,), lambda i: (0,)),
            pl.BlockSpec((_D,), lambda i: (0,)),
        ],
        out_specs=pl.BlockSpec((_BM, _D), lambda i: (i, 0)),
        out_shape=jax.ShapeDtypeStruct((_NP, _D), jnp.float32),
    )(x1, h2s, h3s, agg2, agg3, dinv, b2, b3)


def kernel(x, edge_index, edge_attr, W1, b1, W2, b2, W3, b3):
    src = edge_index[0]
    dst = edge_index[1]
    xp = jnp.pad(x, ((0, _NP - _N), (0, 0)))
    pad_e = _E2 - _E
    dstp = jnp.pad(dst, (0, pad_e))
    attrp = jnp.pad(edge_attr, (0, pad_e))
    srcp = jnp.concatenate(
        [src, jnp.arange(pad_e, dtype=jnp.int32) % _N])

    s1, s2 = _idx_prep(dstp.reshape(_E2 // 128, 128),
                       attrp.reshape(_E2 // 128, 128))
    s1 = s1.reshape(_E2)
    s2 = s2.reshape(_E2)
    s1_3d = s1.reshape(_NW, _NC, _CH)
    s2_3d = s2.reshape(_NW, _NC, _CH)
    s1_4d = s1.reshape(_NSBT, _SB, _CH)
    s2_4d = s2.reshape(_NSBT, _SB, _CH)
    g_4d = srcp.reshape(_NSBT, _SB, _CH)

    ones_c = jnp.ones((_CH,), jnp.float32)
    zeros_r = jnp.zeros((_RPS,), jnp.float32)
    zeros_b = jnp.zeros((64, _D), jnp.float32)

    degp = _deg_kernel(s1_3d, s2_3d, ones_c, zeros_r)
    dinv = _dinv(degp.reshape(2, 2, _NP))

    h1s = _mm_scale(xp, W1, dinv[0])
    agg1 = _agg_kernel(h1s, g_4d, s1_4d, zeros_b)
    x1 = _combine1(xp, h1s, agg1, dinv[0], b1)

    h2s, h3s = _mm2(x1, xp, W2, W3, dinv)
    agg2 = _agg_kernel(h2s, g_4d, s1_4d, zeros_b)
    agg3 = _agg_kernel(h3s, g_4d, s2_4d, zeros_b)
    x2 = _combine2(x1, h2s, h3s, agg2, agg3, dinv, b2, b3)

    return x2[:_N]


# speedup vs baseline: 3.3030x; 1.0512x over previous
import functools

import jax
import jax.numpy as jnp
from jax import lax
from jax.experimental import pallas as pl
from jax.experimental.pallas import tpu as pltpu
from jax.experimental.pallas import tpu_sc as plsc

_N = 10000
_E = 320000
_D = 128
_NP = 10240
_DUMP = _NP - 1
_NW = 32
_E2 = 327680
_EW = _E2 // _NW
_CH = 128
_NC = _EW // _CH
_SB = 8
_NSBT = _E2 // (_SB * _CH)
_NSBW = _NSBT // _NW
_BM = 512
_NB = _NP // _BM
_RPS = _NP // 16

_mesh = plsc.VectorSubcoreMesh(core_axis_name="c", subcore_axis_name="s")


def _idx_prep_body(dst_ref, attr_ref, s1_ref, s2_ref):
    a = attr_ref[...]
    d = dst_ref[...]
    dump = _N + lax.broadcasted_iota(jnp.int32, a.shape, 1)
    s1_ref[...] = jnp.where(a == 1, d, dump)
    s2_ref[...] = jnp.where(a == 2, d, dump)


def _idx_prep(dst2d, attr2d):
    return pl.pallas_call(
        _idx_prep_body,
        out_shape=[jax.ShapeDtypeStruct((_E2 // 128, 128), jnp.int32)] * 2,
    )(dst2d, attr2d)


@functools.partial(
    pl.kernel,
    out_type=jax.ShapeDtypeStruct((2, 2 * _NP), jnp.float32),
    mesh=_mesh,
    scratch_types=[
        pltpu.VMEM((_NC, _CH), jnp.int32),
        pltpu.VMEM((_NC, _CH), jnp.int32),
        pltpu.VMEM((_CH,), jnp.float32),
        pltpu.VMEM((_RPS,), jnp.float32),
        pltpu.VMEM_SHARED((_NP,), jnp.float32),
        pltpu.VMEM_SHARED((_NP,), jnp.float32),
    ],
)
def _deg_kernel(s1_hbm, s2_hbm, ones_hbm, zeros_hbm, degp_hbm,
                s1_v, s2_v, ones_v, zb_v, cnt1_sh, cnt2_sh):
    c = lax.axis_index("c")
    s = lax.axis_index("s")
    w = s * 2 + c

    pltpu.sync_copy(ones_hbm, ones_v)
    pltpu.sync_copy(zeros_hbm, zb_v)
    pltpu.sync_copy(zb_v, cnt1_sh.at[pl.ds(s * _RPS, _RPS)])
    pltpu.sync_copy(zb_v, cnt2_sh.at[pl.ds(s * _RPS, _RPS)])

    plsc.subcore_barrier()

    pltpu.sync_copy(s1_hbm.at[w], s1_v)
    pltpu.sync_copy(s2_hbm.at[w], s2_v)

    @pl.loop(0, _NC)
    def _chunks(j):
        pltpu.sync_copy(ones_v, cnt1_sh.at[s1_v.at[j]], add=True)
        pltpu.sync_copy(ones_v, cnt2_sh.at[s2_v.at[j]], add=True)

    plsc.subcore_barrier()

    for k, cnt_sh in enumerate((cnt1_sh, cnt2_sh)):
        r = s * _RPS
        pltpu.sync_copy(cnt_sh.at[pl.ds(r, _RPS)], zb_v)
        pltpu.sync_copy(zb_v, degp_hbm.at[c, pl.ds(k * _NP + r, _RPS)])


def _dinv_body(degp_ref, dinv_ref):
    dinv_ref[...] = lax.rsqrt(degp_ref[0] + degp_ref[1] + 1.0)


def _dinv(degp):
    return pl.pallas_call(
        _dinv_body,
        out_shape=jax.ShapeDtypeStruct((2, _NP), jnp.float32),
    )(degp)


@functools.partial(
    pl.kernel,
    out_type=jax.ShapeDtypeStruct((2, _NP, _D), jnp.float32),
    mesh=_mesh,
    scratch_types=[
        pltpu.VMEM((_SB, _CH), jnp.int32),
        pltpu.VMEM((_SB, _CH), jnp.int32),
        pltpu.VMEM((2, _CH, _D), jnp.float32),
        pltpu.VMEM_SHARED((_NP, _D), jnp.float32),
        pltpu.SemaphoreType.DMA,
        pltpu.SemaphoreType.DMA,
    ],
)
def _agg_kernel(table_hbm, gidx_hbm, sidx_hbm, z_hbm, out_hbm,
                gidx_v, sidx_v, rows_v, acc_sh, sem0, sem1):
    c = lax.axis_index("c")
    s = lax.axis_index("s")
    w = s * 2 + c
    sems = (sem0, sem1)

    zsrc = rows_v.at[0, pl.ds(0, 64)]
    pltpu.sync_copy(z_hbm, zsrc)
    for k in range(_RPS // 64):
        pltpu.sync_copy(zsrc, acc_sh.at[pl.ds(s * _RPS + k * 64, 64)])

    plsc.subcore_barrier()

    @pl.loop(0, _NSBW)
    def _super(u):
        sb = w * _NSBW + u
        pltpu.sync_copy(gidx_hbm.at[sb], gidx_v)
        pltpu.sync_copy(sidx_hbm.at[sb], sidx_v)

        pltpu.async_copy(table_hbm.at[gidx_v.at[0]], rows_v.at[0], sem0)
        pltpu.async_copy(table_hbm.at[gidx_v.at[1]], rows_v.at[1], sem1)

        @pl.loop(0, _SB, step=2)
        def _chunks(j):
            for b in range(2):
                jj = j + b
                pltpu.make_async_copy(
                    table_hbm.at[gidx_v.at[jj]], rows_v.at[b],
                    sems[b]).wait()
                pltpu.sync_copy(rows_v.at[b], acc_sh.at[sidx_v.at[jj]],
                                add=True)

                @pl.when(jj + 2 < _SB)
                def _():
                    pltpu.async_copy(
                        table_hbm.at[gidx_v.at[jj + 2]], rows_v.at[b],
                        sems[b])

    plsc.subcore_barrier()

    bounce = rows_v.at[0, pl.ds(0, 64)]
    for k in range(_RPS // 64):
        r = s * _RPS + k * 64
        pltpu.sync_copy(acc_sh.at[pl.ds(r, 64)], bounce)
        pltpu.sync_copy(bounce, out_hbm.at[c, pl.ds(r, 64)])


def _mm_scale_body(x_ref, w_ref, dinv_ref, out_ref):
    h = jnp.dot(x_ref[...], w_ref[...], preferred_element_type=jnp.float32)
    out_ref[...] = h * dinv_ref[...][:, None]


def _mm_scale(xp, W, dinv1):
    return pl.pallas_call(
        _mm_scale_body,
        grid=(_NB,),
        in_specs=[
            pl.BlockSpec((_BM, _D), lambda i: (i, 0)),
            pl.BlockSpec((_D, _D), lambda i: (0, 0)),
            pl.BlockSpec((_BM,), lambda i: (i,)),
        ],
        out_specs=pl.BlockSpec((_BM, _D), lambda i: (i, 0)),
        out_shape=jax.ShapeDtypeStruct((_NP, _D), jnp.float32),
    )(xp, W, dinv1)


def _mm2_body(x1_ref, x0_ref, w2_ref, w3_ref, dinv_ref, h2_ref, h3_ref):
    h2 = jnp.dot(x1_ref[...], w2_ref[...], preferred_element_type=jnp.float32)
    h3 = jnp.dot(x0_ref[...], w3_ref[...], preferred_element_type=jnp.float32)
    h2_ref[...] = h2 * dinv_ref[0][:, None]
    h3_ref[...] = h3 * dinv_ref[1][:, None]


def _mm2(x1, xp, W2, W3, dinv):
    return pl.pallas_call(
        _mm2_body,
        grid=(_NB,),
        in_specs=[
            pl.BlockSpec((_BM, _D), lambda i: (i, 0)),
            pl.BlockSpec((_BM, _D), lambda i: (i, 0)),
            pl.BlockSpec((_D, _D), lambda i: (0, 0)),
            pl.BlockSpec((_D, _D), lambda i: (0, 0)),
            pl.BlockSpec((2, _BM), lambda i: (0, i)),
        ],
        out_specs=[pl.BlockSpec((_BM, _D), lambda i: (i, 0))] * 2,
        out_shape=[jax.ShapeDtypeStruct((_NP, _D), jnp.float32)] * 2,
    )(x1, xp, W2, W3, dinv)


def _combine1_body(x_ref, hs_ref, aggp_ref, dinv_ref, b_ref, out_ref):
    agg = aggp_ref[0] + aggp_ref[1] + hs_ref[...]
    o = agg * dinv_ref[...][:, None] + b_ref[...][None, :]
    t = x_ref[...] + jnp.maximum(o, 0.0)
    nrm = jnp.sqrt(jnp.sum(t * t, axis=-1, keepdims=True))
    out_ref[...] = t / jnp.maximum(nrm, 1e-12)


def _combine1(xp, h1s, aggp, dinv1, b1):
    return pl.pallas_call(
        _combine1_body,
        grid=(_NB,),
        in_specs=[
            pl.BlockSpec((_BM, _D), lambda i: (i, 0)),
            pl.BlockSpec((_BM, _D), lambda i: (i, 0)),
            pl.BlockSpec((2, _BM, _D), lambda i: (0, i, 0)),
            pl.BlockSpec((_BM,), lambda i: (i,)),
            pl.BlockSpec((_D,), lambda i: (0,)),
        ],
        out_specs=pl.BlockSpec((_BM, _D), lambda i: (i, 0)),
        out_shape=jax.ShapeDtypeStruct((_NP, _D), jnp.float32),
    )(xp, h1s, aggp, dinv1, b1)


def _combine2_body(x1_ref, h2_ref, h3_ref, agg2_ref, agg3_ref, dinv_ref,
                   b2_ref, b3_ref, out_ref):
    a2 = (agg2_ref[0] + agg2_ref[1] + h2_ref[...]) * dinv_ref[0][:, None] \
        + b2_ref[...][None, :]
    a3 = (agg3_ref[0] + agg3_ref[1] + h3_ref[...]) * dinv_ref[1][:, None] \
        + b3_ref[...][None, :]
    t = x1_ref[...] + jnp.maximum(a2 + a3, 0.0)
    nrm = jnp.sqrt(jnp.sum(t * t, axis=-1, keepdims=True))
    out_ref[...] = t / jnp.maximum(nrm, 1e-12)


def _combine2(x1, h2s, h3s, agg2, agg3, dinv, b2, b3):
    return pl.pallas_call(
        _combine2_body,
        grid=(_NB,),
        in_specs=[
            pl.BlockSpec((_BM, _D), lambda i: (i, 0)),
            pl.BlockSpec((_BM, _D), lambda i: (i, 0)),
            pl.BlockSpec((_BM, _D), lambda i: (i, 0)),
            pl.BlockSpec((2, _BM, _D), lambda i: (0, i, 0)),
            pl.BlockSpec((2, _BM, _D), lambda i: (0, i, 0)),
            pl.BlockSpec((2, _BM), lambda i: (0, i)),
            pl.BlockSpec((_D,), lambda i: (0,)),
            pl.BlockSpec((_D,), lambda i: (0,)),
        ],
        out_specs=pl.BlockSpec((_BM, _D), lambda i: (i, 0)),
        out_shape=jax.ShapeDtypeStruct((_NP, _D), jnp.float32),
    )(x1, h2s, h3s, agg2, agg3, dinv, b2, b3)


def kernel(x, edge_index, edge_attr, W1, b1, W2, b2, W3, b3):
    src = edge_index[0]
    dst = edge_index[1]
    xp = jnp.pad(x, ((0, _NP - _N), (0, 0)))
    pad_e = _E2 - _E
    dstp = jnp.pad(dst, (0, pad_e))
    attrp = jnp.pad(edge_attr, (0, pad_e))
    srcp = jnp.concatenate(
        [src, jnp.arange(pad_e, dtype=jnp.int32) % _N])

    s1, s2 = _idx_prep(dstp.reshape(_E2 // 128, 128),
                       attrp.reshape(_E2 // 128, 128))
    s1 = s1.reshape(_E2)
    s2 = s2.reshape(_E2)
    s1_3d = s1.reshape(_NW, _NC, _CH)
    s2_3d = s2.reshape(_NW, _NC, _CH)
    s1_4d = s1.reshape(_NSBT, _SB, _CH)
    s2_4d = s2.reshape(_NSBT, _SB, _CH)
    g_4d = srcp.reshape(_NSBT, _SB, _CH)

    ones_c = jnp.ones((_CH,), jnp.float32)
    zeros_r = jnp.zeros((_RPS,), jnp.float32)
    zeros_b = jnp.zeros((64, _D), jnp.float32)

    degp = _deg_kernel(s1_3d, s2_3d, ones_c, zeros_r)
    dinv = _dinv(degp.reshape(2, 2, _NP))

    h1s, h3s = _mm2(xp, xp, W1, W3, dinv)
    agg1 = _agg_kernel(h1s, g_4d, s1_4d, zeros_b)
    agg3 = _agg_kernel(h3s, g_4d, s2_4d, zeros_b)
    x1 = _combine1(xp, h1s, agg1, dinv[0], b1)

    h2s = _mm_scale(x1, W2, dinv[0])
    agg2 = _agg_kernel(h2s, g_4d, s1_4d, zeros_b)
    x2 = _combine2(x1, h2s, h3s, agg2, agg3, dinv, b2, b3)

    return x2[:_N]


# speedup vs baseline: 4.2458x; 1.2855x over previous
import functools

import jax
import jax.numpy as jnp
from jax import lax
from jax.experimental import pallas as pl
from jax.experimental.pallas import tpu as pltpu
from jax.experimental.pallas import tpu_sc as plsc

_N = 10000
_E = 320000
_D = 128
_NP = 10240
_DUMP = _NP - 1
_NW = 32
_E2 = 327680
_EW = _E2 // _NW
_CH = 128
_NC = _EW // _CH
_SB = 8
_NSBT = _E2 // (_SB * _CH)
_NSBW = _NSBT // _NW
_BM = 512
_NB = _NP // _BM
_RPS = _NP // 16

_mesh = plsc.VectorSubcoreMesh(core_axis_name="c", subcore_axis_name="s")


def _idx_prep_body(dst_ref, attr_ref, s1_ref, s2_ref):
    a = attr_ref[...]
    d = dst_ref[...]
    dump = _N + lax.broadcasted_iota(jnp.int32, a.shape, 1)
    s1_ref[...] = jnp.where(a == 1, d, dump)
    s2_ref[...] = jnp.where(a == 2, d, dump)


def _idx_prep(dst2d, attr2d):
    return pl.pallas_call(
        _idx_prep_body,
        out_shape=[jax.ShapeDtypeStruct((_E2 // 128, 128), jnp.int32)] * 2,
    )(dst2d, attr2d)


@functools.partial(
    pl.kernel,
    out_type=jax.ShapeDtypeStruct((2, 2 * _NP), jnp.float32),
    mesh=_mesh,
    scratch_types=[
        pltpu.VMEM((_NC, _CH), jnp.int32),
        pltpu.VMEM((_NC, _CH), jnp.int32),
        pltpu.VMEM((_CH,), jnp.float32),
        pltpu.VMEM((_RPS,), jnp.float32),
        pltpu.VMEM_SHARED((_NP,), jnp.float32),
        pltpu.VMEM_SHARED((_NP,), jnp.float32),
    ],
)
def _deg_kernel(s1_hbm, s2_hbm, ones_hbm, zeros_hbm, degp_hbm,
                s1_v, s2_v, ones_v, zb_v, cnt1_sh, cnt2_sh):
    c = lax.axis_index("c")
    s = lax.axis_index("s")
    w = s * 2 + c

    pltpu.sync_copy(ones_hbm, ones_v)
    pltpu.sync_copy(zeros_hbm, zb_v)
    pltpu.sync_copy(zb_v, cnt1_sh.at[pl.ds(s * _RPS, _RPS)])
    pltpu.sync_copy(zb_v, cnt2_sh.at[pl.ds(s * _RPS, _RPS)])

    plsc.subcore_barrier()

    pltpu.sync_copy(s1_hbm.at[w], s1_v)
    pltpu.sync_copy(s2_hbm.at[w], s2_v)

    @pl.loop(0, _NC)
    def _chunks(j):
        pltpu.sync_copy(ones_v, cnt1_sh.at[s1_v.at[j]], add=True)
        pltpu.sync_copy(ones_v, cnt2_sh.at[s2_v.at[j]], add=True)

    plsc.subcore_barrier()

    for k, cnt_sh in enumerate((cnt1_sh, cnt2_sh)):
        r = s * _RPS
        pltpu.sync_copy(cnt_sh.at[pl.ds(r, _RPS)], zb_v)
        pltpu.sync_copy(zb_v, degp_hbm.at[c, pl.ds(k * _NP + r, _RPS)])


@functools.partial(
    pl.kernel,
    out_type=[
        jax.ShapeDtypeStruct((_NW, _EW), jnp.int32),
        jax.ShapeDtypeStruct((_NW, _EW), jnp.int32),
        jax.ShapeDtypeStruct((_NW, _EW), jnp.int32),
        jax.ShapeDtypeStruct((_NW, _EW), jnp.int32),
        jax.ShapeDtypeStruct((_NW * 16,), jnp.int32),
    ],
    mesh=_mesh,
    scratch_types=[
        pltpu.VMEM((_EW,), jnp.int32),
        pltpu.VMEM((_EW,), jnp.int32),
        pltpu.VMEM((_EW,), jnp.int32),
        pltpu.VMEM((_EW + 16,), jnp.int32),
        pltpu.VMEM((_EW + 16,), jnp.int32),
        pltpu.VMEM((_EW + 16,), jnp.int32),
        pltpu.VMEM((_EW + 16,), jnp.int32),
        pltpu.VMEM((16,), jnp.int32),
    ],
    compiler_params=pltpu.CompilerParams(needs_layout_passes=False),
)
def _compact_kernel(s1_hbm, s2_hbm, src_hbm, padg_hbm, padd_hbm,
                    cg1_hbm, cd1_hbm, cg2_hbm, cd2_hbm, cnt_hbm,
                    s1_v, s2_v, src_v, cg1_v, cd1_v, cg2_v, cd2_v, cnt_v):
    c = lax.axis_index("c")
    s = lax.axis_index("s")
    w = s * 2 + c

    pltpu.sync_copy(s1_hbm.at[w], s1_v)
    pltpu.sync_copy(s2_hbm.at[w], s2_v)
    pltpu.sync_copy(src_hbm.at[w], src_v)
    pltpu.sync_copy(padg_hbm, cg1_v.at[pl.ds(0, _EW)])
    pltpu.sync_copy(padd_hbm, cd1_v.at[pl.ds(0, _EW)])
    pltpu.sync_copy(padg_hbm, cg2_v.at[pl.ds(0, _EW)])
    pltpu.sync_copy(padd_hbm, cd2_v.at[pl.ds(0, _EW)])

    @pl.loop(0, _EW // 16, init_carry=(jnp.int32(0), jnp.int32(0)))
    def _compact(g, carry):
        o1, o2 = carry
        vs1 = s1_v[pl.ds(g * 16, 16)]
        vs2 = s2_v[pl.ds(g * 16, 16)]
        vsrc = src_v[pl.ds(g * 16, 16)]
        m1 = vs1 < _N
        m2 = vs2 < _N
        plsc.store_compressed(cd1_v.at[pl.ds(o1, 16)], vs1, mask=m1)
        plsc.store_compressed(cg1_v.at[pl.ds(o1, 16)], vsrc, mask=m1)
        plsc.store_compressed(cd2_v.at[pl.ds(o2, 16)], vs2, mask=m2)
        plsc.store_compressed(cg2_v.at[pl.ds(o2, 16)], vsrc, mask=m2)
        n1 = jnp.max(plsc.all_reduce_population_count(m1))
        n2 = jnp.max(plsc.all_reduce_population_count(m2))
        return o1 + n1, o2 + n2

    o1, o2 = _compact
    lane = lax.iota(jnp.int32, 16)
    cnt_v[pl.ds(0, 16)] = jnp.where(lane == 0, o1,
                                    jnp.where(lane == 1, o2, 0))

    pltpu.sync_copy(cg1_v.at[pl.ds(0, _EW)], cg1_hbm.at[w])
    pltpu.sync_copy(cd1_v.at[pl.ds(0, _EW)], cd1_hbm.at[w])
    pltpu.sync_copy(cg2_v.at[pl.ds(0, _EW)], cg2_hbm.at[w])
    pltpu.sync_copy(cd2_v.at[pl.ds(0, _EW)], cd2_hbm.at[w])
    pltpu.sync_copy(cnt_v, cnt_hbm.at[pl.ds(w * 16, 16)])


def _dinv_body(degp_ref, dinv_ref):
    dinv_ref[...] = lax.rsqrt(degp_ref[0] + degp_ref[1] + 1.0)


def _dinv(degp):
    return pl.pallas_call(
        _dinv_body,
        out_shape=jax.ShapeDtypeStruct((2, _NP), jnp.float32),
    )(degp)


@functools.partial(
    pl.kernel,
    out_type=jax.ShapeDtypeStruct((2, _NP, _D), jnp.float32),
    mesh=_mesh,
    scratch_types=[
        pltpu.VMEM((_SB, _CH), jnp.int32),
        pltpu.VMEM((_SB, _CH), jnp.int32),
        pltpu.VMEM((2, _CH, _D), jnp.float32),
        pltpu.VMEM((_NW + 16,), jnp.int32),
        pltpu.VMEM_SHARED((_NP, _D), jnp.float32),
        pltpu.SemaphoreType.DMA,
        pltpu.SemaphoreType.DMA,
    ],
)
def _agg_kernel(table_hbm, gidx_hbm, sidx_hbm, z_hbm, cnt_hbm, out_hbm,
                gidx_v, sidx_v, rows_v, cnt_v, acc_sh, sem0, sem1):
    c = lax.axis_index("c")
    s = lax.axis_index("s")
    w = s * 2 + c
    sems = (sem0, sem1)

    pltpu.sync_copy(cnt_hbm, cnt_v)

    zsrc = rows_v.at[0, pl.ds(0, 64)]
    pltpu.sync_copy(z_hbm, zsrc)
    for k in range(_RPS // 64):
        pltpu.sync_copy(zsrc, acc_sh.at[pl.ds(s * _RPS + k * 64, 64)])

    plsc.subcore_barrier()

    cvec = cnt_v[pl.ds(w, 16)]
    nsb = (cvec[0] + (_SB * _CH - 1)) // (_SB * _CH)

    @pl.loop(0, nsb)
    def _super(u):
        sb = w * _NSBW + u
        pltpu.sync_copy(gidx_hbm.at[sb], gidx_v)
        pltpu.sync_copy(sidx_hbm.at[sb], sidx_v)

        pltpu.async_copy(table_hbm.at[gidx_v.at[0]], rows_v.at[0], sem0)
        pltpu.async_copy(table_hbm.at[gidx_v.at[1]], rows_v.at[1], sem1)

        @pl.loop(0, _SB, step=2)
        def _chunks(j):
            for b in range(2):
                jj = j + b
                pltpu.make_async_copy(
                    table_hbm.at[gidx_v.at[jj]], rows_v.at[b],
                    sems[b]).wait()
                pltpu.sync_copy(rows_v.at[b], acc_sh.at[sidx_v.at[jj]],
                                add=True)

                @pl.when(jj + 2 < _SB)
                def _():
                    pltpu.async_copy(
                        table_hbm.at[gidx_v.at[jj + 2]], rows_v.at[b],
                        sems[b])

    plsc.subcore_barrier()

    bounce = rows_v.at[0, pl.ds(0, 64)]
    for k in range(_RPS // 64):
        r = s * _RPS + k * 64
        pltpu.sync_copy(acc_sh.at[pl.ds(r, 64)], bounce)
        pltpu.sync_copy(bounce, out_hbm.at[c, pl.ds(r, 64)])


def _mm_scale_body(x_ref, w_ref, dinv_ref, out_ref):
    h = jnp.dot(x_ref[...], w_ref[...], preferred_element_type=jnp.float32)
    out_ref[...] = h * dinv_ref[...][:, None]


def _mm_scale(xp, W, dinv1):
    return pl.pallas_call(
        _mm_scale_body,
        grid=(_NB,),
        in_specs=[
            pl.BlockSpec((_BM, _D), lambda i: (i, 0)),
            pl.BlockSpec((_D, _D), lambda i: (0, 0)),
            pl.BlockSpec((_BM,), lambda i: (i,)),
        ],
        out_specs=pl.BlockSpec((_BM, _D), lambda i: (i, 0)),
        out_shape=jax.ShapeDtypeStruct((_NP, _D), jnp.float32),
    )(xp, W, dinv1)


def _mm2_body(x1_ref, x0_ref, w2_ref, w3_ref, dinv_ref, h2_ref, h3_ref):
    h2 = jnp.dot(x1_ref[...], w2_ref[...], preferred_element_type=jnp.float32)
    h3 = jnp.dot(x0_ref[...], w3_ref[...], preferred_element_type=jnp.float32)
    h2_ref[...] = h2 * dinv_ref[0][:, None]
    h3_ref[...] = h3 * dinv_ref[1][:, None]


def _mm2(x1, xp, W2, W3, dinv):
    return pl.pallas_call(
        _mm2_body,
        grid=(_NB,),
        in_specs=[
            pl.BlockSpec((_BM, _D), lambda i: (i, 0)),
            pl.BlockSpec((_BM, _D), lambda i: (i, 0)),
            pl.BlockSpec((_D, _D), lambda i: (0, 0)),
            pl.BlockSpec((_D, _D), lambda i: (0, 0)),
            pl.BlockSpec((2, _BM), lambda i: (0, i)),
        ],
        out_specs=[pl.BlockSpec((_BM, _D), lambda i: (i, 0))] * 2,
        out_shape=[jax.ShapeDtypeStruct((_NP, _D), jnp.float32)] * 2,
    )(x1, xp, W2, W3, dinv)


def _combine1_body(x_ref, hs_ref, aggp_ref, dinv_ref, b_ref, out_ref):
    agg = aggp_ref[0] + aggp_ref[1] + hs_ref[...]
    o = agg * dinv_ref[...][:, None] + b_ref[...][None, :]
    t = x_ref[...] + jnp.maximum(o, 0.0)
    nrm = jnp.sqrt(jnp.sum(t * t, axis=-1, keepdims=True))
    out_ref[...] = t / jnp.maximum(nrm, 1e-12)


def _combine1(xp, h1s, aggp, dinv1, b1):
    return pl.pallas_call(
        _combine1_body,
        grid=(_NB,),
        in_specs=[
            pl.BlockSpec((_BM, _D), lambda i: (i, 0)),
            pl.BlockSpec((_BM, _D), lambda i: (i, 0)),
            pl.BlockSpec((2, _BM, _D), lambda i: (0, i, 0)),
            pl.BlockSpec((_BM,), lambda i: (i,)),
            pl.BlockSpec((_D,), lambda i: (0,)),
        ],
        out_specs=pl.BlockSpec((_BM, _D), lambda i: (i, 0)),
        out_shape=jax.ShapeDtypeStruct((_NP, _D), jnp.float32),
    )(xp, h1s, aggp, dinv1, b1)


def _combine2_body(x1_ref, h2_ref, h3_ref, agg2_ref, agg3_ref, dinv_ref,
                   b2_ref, b3_ref, out_ref):
    a2 = (agg2_ref[0] + agg2_ref[1] + h2_ref[...]) * dinv_ref[0][:, None] \
        + b2_ref[...][None, :]
    a3 = (agg3_ref[0] + agg3_ref[1] + h3_ref[...]) * dinv_ref[1][:, None] \
        + b3_ref[...][None, :]
    t = x1_ref[...] + jnp.maximum(a2 + a3, 0.0)
    nrm = jnp.sqrt(jnp.sum(t * t, axis=-1, keepdims=True))
    out_ref[...] = t / jnp.maximum(nrm, 1e-12)


def _combine2(x1, h2s, h3s, agg2, agg3, dinv, b2, b3):
    return pl.pallas_call(
        _combine2_body,
        grid=(_NB,),
        in_specs=[
            pl.BlockSpec((_BM, _D), lambda i: (i, 0)),
            pl.BlockSpec((_BM, _D), lambda i: (i, 0)),
            pl.BlockSpec((_BM, _D), lambda i: (i, 0)),
            pl.BlockSpec((2, _BM, _D), lambda i: (0, i, 0)),
            pl.BlockSpec((2, _BM, _D), lambda i: (0, i, 0)),
            pl.BlockSpec((2, _BM), lambda i: (0, i)),
            pl.BlockSpec((_D,), lambda i: (0,)),
            pl.BlockSpec((_D,), lambda i: (0,)),
        ],
        out_specs=pl.BlockSpec((_BM, _D), lambda i: (i, 0)),
        out_shape=jax.ShapeDtypeStruct((_NP, _D), jnp.float32),
    )(x1, h2s, h3s, agg2, agg3, dinv, b2, b3)


def kernel(x, edge_index, edge_attr, W1, b1, W2, b2, W3, b3):
    src = edge_index[0]
    dst = edge_index[1]
    xp = jnp.pad(x, ((0, _NP - _N), (0, 0)))
    pad_e = _E2 - _E
    dstp = jnp.pad(dst, (0, pad_e))
    attrp = jnp.pad(edge_attr, (0, pad_e))
    srcp = jnp.concatenate(
        [src, jnp.arange(pad_e, dtype=jnp.int32) % _N])

    s1, s2 = _idx_prep(dstp.reshape(_E2 // 128, 128),
                       attrp.reshape(_E2 // 128, 128))
    s1 = s1.reshape(_E2)
    s2 = s2.reshape(_E2)
    s1_3d = s1.reshape(_NW, _NC, _CH)
    s2_3d = s2.reshape(_NW, _NC, _CH)

    ones_c = jnp.ones((_CH,), jnp.float32)
    zeros_r = jnp.zeros((_RPS,), jnp.float32)
    zeros_b = jnp.zeros((64, _D), jnp.float32)
    spread = jnp.arange(_EW, dtype=jnp.int32)
    padg = spread % _N
    padd = _N + (spread % 128)

    degp = _deg_kernel(s1_3d, s2_3d, ones_c, zeros_r)
    dinv = _dinv(degp.reshape(2, 2, _NP))

    cg1, cd1, cg2, cd2, cnts = _compact_kernel(
        s1.reshape(_NW, _EW), s2.reshape(_NW, _EW),
        srcp.reshape(_NW, _EW), padg, padd)
    cnt1 = jnp.pad(cnts.reshape(_NW, 16)[:, 0], (0, 16))
    cnt2 = jnp.pad(cnts.reshape(_NW, 16)[:, 1], (0, 16))
    cg1 = cg1.reshape(_NSBT, _SB, _CH)
    cd1 = cd1.reshape(_NSBT, _SB, _CH)
    cg2 = cg2.reshape(_NSBT, _SB, _CH)
    cd2 = cd2.reshape(_NSBT, _SB, _CH)

    h1s, h3s = _mm2(xp, xp, W1, W3, dinv)
    agg1 = _agg_kernel(h1s, cg1, cd1, zeros_b, cnt1)
    agg3 = _agg_kernel(h3s, cg2, cd2, zeros_b, cnt2)
    x1 = _combine1(xp, h1s, agg1, dinv[0], b1)

    h2s = _mm_scale(x1, W2, dinv[0])
    agg2 = _agg_kernel(h2s, cg1, cd1, zeros_b, cnt1)
    x2 = _combine2(x1, h2s, h3s, agg2, agg3, dinv, b2, b3)

    return x2[:_N]


# speedup vs baseline: 4.2480x; 1.0005x over previous
import functools

import jax
import jax.numpy as jnp
from jax import lax
from jax.experimental import pallas as pl
from jax.experimental.pallas import tpu as pltpu
from jax.experimental.pallas import tpu_sc as plsc

_N = 10000
_E = 320000
_D = 128
_NP = 10240
_NW = 32
_E2 = 327680
_EW = _E2 // _NW
_CH = 128
_NC = _EW // _CH
_SB = 8
_NSBT = _E2 // (_SB * _CH)
_NSBW = _NSBT // _NW
_BM = 512
_NB = _NP // _BM
_RPS = _NP // 16

_mesh = plsc.VectorSubcoreMesh(core_axis_name="c", subcore_axis_name="s")


def _idx_prep_body(dst_ref, attr_ref, s1_ref, s2_ref):
    a = attr_ref[...]
    d = dst_ref[...]
    dump = _N + lax.broadcasted_iota(jnp.int32, a.shape, 1)
    s1_ref[...] = jnp.where(a == 1, d, dump)
    s2_ref[...] = jnp.where(a == 2, d, dump)


def _idx_prep(dst2d, attr2d):
    return pl.pallas_call(
        _idx_prep_body,
        out_shape=[jax.ShapeDtypeStruct((_E2 // 128, 128), jnp.int32)] * 2,
    )(dst2d, attr2d)


@functools.partial(
    pl.kernel,
    out_type=jax.ShapeDtypeStruct((2, 2 * _NP), jnp.float32),
    mesh=_mesh,
    scratch_types=[
        pltpu.VMEM((_NC, _CH), jnp.int32),
        pltpu.VMEM((_NC, _CH), jnp.int32),
        pltpu.VMEM((_CH,), jnp.float32),
        pltpu.VMEM((_RPS,), jnp.float32),
        pltpu.VMEM_SHARED((_NP,), jnp.float32),
        pltpu.VMEM_SHARED((_NP,), jnp.float32),
    ],
)
def _deg_kernel(s1_hbm, s2_hbm, ones_hbm, zeros_hbm, degp_hbm,
                s1_v, s2_v, ones_v, zb_v, cnt1_sh, cnt2_sh):
    c = lax.axis_index("c")
    s = lax.axis_index("s")
    w = s * 2 + c

    pltpu.sync_copy(ones_hbm, ones_v)
    pltpu.sync_copy(zeros_hbm, zb_v)
    pltpu.sync_copy(zb_v, cnt1_sh.at[pl.ds(s * _RPS, _RPS)])
    pltpu.sync_copy(zb_v, cnt2_sh.at[pl.ds(s * _RPS, _RPS)])

    plsc.subcore_barrier()

    pltpu.sync_copy(s1_hbm.at[w], s1_v)
    pltpu.sync_copy(s2_hbm.at[w], s2_v)

    @pl.loop(0, _NC)
    def _chunks(j):
        pltpu.sync_copy(ones_v, cnt1_sh.at[s1_v.at[j]], add=True)
        pltpu.sync_copy(ones_v, cnt2_sh.at[s2_v.at[j]], add=True)

    plsc.subcore_barrier()

    for k, cnt_sh in enumerate((cnt1_sh, cnt2_sh)):
        r = s * _RPS
        pltpu.sync_copy(cnt_sh.at[pl.ds(r, _RPS)], zb_v)
        pltpu.sync_copy(zb_v, degp_hbm.at[c, pl.ds(k * _NP + r, _RPS)])


@functools.partial(
    pl.kernel,
    out_type=[
        jax.ShapeDtypeStruct((_NW, _EW), jnp.int32),
        jax.ShapeDtypeStruct((_NW, _EW), jnp.int32),
        jax.ShapeDtypeStruct((_NW, _EW), jnp.int32),
        jax.ShapeDtypeStruct((_NW, _EW), jnp.int32),
        jax.ShapeDtypeStruct((_NW * 16,), jnp.int32),
    ],
    mesh=_mesh,
    scratch_types=[
        pltpu.VMEM((_EW,), jnp.int32),
        pltpu.VMEM((_EW,), jnp.int32),
        pltpu.VMEM((_EW,), jnp.int32),
        pltpu.VMEM((_EW + 16,), jnp.int32),
        pltpu.VMEM((_EW + 16,), jnp.int32),
        pltpu.VMEM((_EW + 16,), jnp.int32),
        pltpu.VMEM((_EW + 16,), jnp.int32),
        pltpu.VMEM((16,), jnp.int32),
    ],
    compiler_params=pltpu.CompilerParams(needs_layout_passes=False),
)
def _compact_kernel(s1_hbm, s2_hbm, src_hbm, padg_hbm, padd_hbm,
                    cg1_hbm, cd1_hbm, cg2_hbm, cd2_hbm, cnt_hbm,
                    s1_v, s2_v, src_v, cg1_v, cd1_v, cg2_v, cd2_v, cnt_v):
    c = lax.axis_index("c")
    s = lax.axis_index("s")
    w = s * 2 + c

    pltpu.sync_copy(s1_hbm.at[w], s1_v)
    pltpu.sync_copy(s2_hbm.at[w], s2_v)
    pltpu.sync_copy(src_hbm.at[w], src_v)
    pltpu.sync_copy(padg_hbm, cg1_v.at[pl.ds(0, _EW)])
    pltpu.sync_copy(padd_hbm, cd1_v.at[pl.ds(0, _EW)])
    pltpu.sync_copy(padg_hbm, cg2_v.at[pl.ds(0, _EW)])
    pltpu.sync_copy(padd_hbm, cd2_v.at[pl.ds(0, _EW)])

    @pl.loop(0, _EW // 16, init_carry=(jnp.int32(0), jnp.int32(0)))
    def _compact(g, carry):
        o1, o2 = carry
        vs1 = s1_v[pl.ds(g * 16, 16)]
        vs2 = s2_v[pl.ds(g * 16, 16)]
        vsrc = src_v[pl.ds(g * 16, 16)]
        m1 = vs1 < _N
        m2 = vs2 < _N
        plsc.store_compressed(cd1_v.at[pl.ds(o1, 16)], vs1, mask=m1)
        plsc.store_compressed(cg1_v.at[pl.ds(o1, 16)], vsrc, mask=m1)
        plsc.store_compressed(cd2_v.at[pl.ds(o2, 16)], vs2, mask=m2)
        plsc.store_compressed(cg2_v.at[pl.ds(o2, 16)], vsrc, mask=m2)
        n1 = jnp.max(plsc.all_reduce_population_count(m1))
        n2 = jnp.max(plsc.all_reduce_population_count(m2))
        return o1 + n1, o2 + n2

    o1, o2 = _compact
    lane = lax.iota(jnp.int32, 16)
    cnt_v[pl.ds(0, 16)] = jnp.where(lane == 0, o1,
                                    jnp.where(lane == 1, o2, 0))

    pltpu.sync_copy(cg1_v.at[pl.ds(0, _EW)], cg1_hbm.at[w])
    pltpu.sync_copy(cd1_v.at[pl.ds(0, _EW)], cd1_hbm.at[w])
    pltpu.sync_copy(cg2_v.at[pl.ds(0, _EW)], cg2_hbm.at[w])
    pltpu.sync_copy(cd2_v.at[pl.ds(0, _EW)], cd2_hbm.at[w])
    pltpu.sync_copy(cnt_v, cnt_hbm.at[pl.ds(w * 16, 16)])


def _dinv_body(degp_ref, dinv_ref):
    dinv_ref[...] = lax.rsqrt(degp_ref[0] + degp_ref[1] + 1.0)


def _dinv(degp):
    return pl.pallas_call(
        _dinv_body,
        out_shape=jax.ShapeDtypeStruct((2, _NP), jnp.float32),
    )(degp)


@functools.partial(
    pl.kernel,
    out_type=jax.ShapeDtypeStruct((2, _NP, _D), jnp.float32),
    mesh=_mesh,
    scratch_types=[
        pltpu.VMEM((_SB, _CH), jnp.int32),
        pltpu.VMEM((_SB, _CH), jnp.int32),
        pltpu.VMEM((2, _CH, _D), jnp.float32),
        pltpu.VMEM((_NW + 16,), jnp.int32),
        pltpu.VMEM_SHARED((_NP, _D), jnp.float32),
        pltpu.SemaphoreType.DMA,
        pltpu.SemaphoreType.DMA,
    ],
)
def _agg_kernel(table_hbm, gidx_hbm, sidx_hbm, z_hbm, cnt_hbm, out_hbm,
                gidx_v, sidx_v, rows_v, cnt_v, acc_sh, sem0, sem1):
    c = lax.axis_index("c")
    s = lax.axis_index("s")
    w = s * 2 + c
    sems = (sem0, sem1)

    pltpu.sync_copy(cnt_hbm, cnt_v)

    zsrc = rows_v.at[0, pl.ds(0, 64)]
    pltpu.sync_copy(z_hbm, zsrc)
    for k in range(_RPS // 64):
        pltpu.sync_copy(zsrc, acc_sh.at[pl.ds(s * _RPS + k * 64, 64)])

    plsc.subcore_barrier()

    cvec = cnt_v[pl.ds(w, 16)]
    nsb = (cvec[0] + (_SB * _CH - 1)) // (_SB * _CH)

    @pl.loop(0, nsb)
    def _super(u):
        sb = w * _NSBW + u
        pltpu.sync_copy(gidx_hbm.at[sb], gidx_v)
        pltpu.sync_copy(sidx_hbm.at[sb], sidx_v)

        pltpu.async_copy(table_hbm.at[gidx_v.at[0]], rows_v.at[0], sem0)
        pltpu.async_copy(table_hbm.at[gidx_v.at[1]], rows_v.at[1], sem1)

        @pl.loop(0, _SB, step=2)
        def _chunks(j):
            for b in range(2):
                jj = j + b
                pltpu.make_async_copy(
                    table_hbm.at[gidx_v.at[jj]], rows_v.at[b],
                    sems[b]).wait()
                pltpu.sync_copy(rows_v.at[b], acc_sh.at[sidx_v.at[jj]],
                                add=True)

                @pl.when(jj + 2 < _SB)
                def _():
                    pltpu.async_copy(
                        table_hbm.at[gidx_v.at[jj + 2]], rows_v.at[b],
                        sems[b])

    plsc.subcore_barrier()

    bounce = rows_v.at[0, pl.ds(0, 64)]
    for k in range(_RPS // 64):
        r = s * _RPS + k * 64
        pltpu.sync_copy(acc_sh.at[pl.ds(r, 64)], bounce)
        pltpu.sync_copy(bounce, out_hbm.at[c, pl.ds(r, 64)])


def _mm_scale_body(x_ref, w_ref, dinv_ref, out_ref):
    h = jnp.dot(x_ref[...], w_ref[...], preferred_element_type=jnp.float32)
    out_ref[...] = h * dinv_ref[...][:, None]


def _mm_scale(xp, W, dinv1):
    return pl.pallas_call(
        _mm_scale_body,
        grid=(_NB,),
        in_specs=[
            pl.BlockSpec((_BM, _D), lambda i: (i, 0)),
            pl.BlockSpec((_D, _D), lambda i: (0, 0)),
            pl.BlockSpec((_BM,), lambda i: (i,)),
        ],
        out_specs=pl.BlockSpec((_BM, _D), lambda i: (i, 0)),
        out_shape=jax.ShapeDtypeStruct((_NP, _D), jnp.float32),
    )(xp, W, dinv1)


def _mm2_body(x1_ref, x0_ref, w2_ref, w3_ref, dinv_ref, h2_ref, h3_ref):
    h2 = jnp.dot(x1_ref[...], w2_ref[...], preferred_element_type=jnp.float32)
    h3 = jnp.dot(x0_ref[...], w3_ref[...], preferred_element_type=jnp.float32)
    h2_ref[...] = h2 * dinv_ref[0][:, None]
    h3_ref[...] = h3 * dinv_ref[1][:, None]


def _mm2(x1, xp, W2, W3, dinv):
    return pl.pallas_call(
        _mm2_body,
        grid=(_NB,),
        in_specs=[
            pl.BlockSpec((_BM, _D), lambda i: (i, 0)),
            pl.BlockSpec((_BM, _D), lambda i: (i, 0)),
            pl.BlockSpec((_D, _D), lambda i: (0, 0)),
            pl.BlockSpec((_D, _D), lambda i: (0, 0)),
            pl.BlockSpec((2, _BM), lambda i: (0, i)),
        ],
        out_specs=[pl.BlockSpec((_BM, _D), lambda i: (i, 0))] * 2,
        out_shape=[jax.ShapeDtypeStruct((_NP, _D), jnp.float32)] * 2,
    )(x1, xp, W2, W3, dinv)


def _combine1_body(x_ref, hs_ref, aggp_ref, dinv_ref, b_ref, out_ref):
    agg = aggp_ref[0] + aggp_ref[1] + hs_ref[...]
    o = agg * dinv_ref[...][:, None] + b_ref[...][None, :]
    t = x_ref[...] + jnp.maximum(o, 0.0)
    nrm = jnp.sqrt(jnp.sum(t * t, axis=-1, keepdims=True))
    out_ref[...] = t / jnp.maximum(nrm, 1e-12)


def _combine1(xp, h1s, aggp, dinv1, b1):
    return pl.pallas_call(
        _combine1_body,
        grid=(_NB,),
        in_specs=[
            pl.BlockSpec((_BM, _D), lambda i: (i, 0)),
            pl.BlockSpec((_BM, _D), lambda i: (i, 0)),
            pl.BlockSpec((2, _BM, _D), lambda i: (0, i, 0)),
            pl.BlockSpec((_BM,), lambda i: (i,)),
            pl.BlockSpec((_D,), lambda i: (0,)),
        ],
        out_specs=pl.BlockSpec((_BM, _D), lambda i: (i, 0)),
        out_shape=jax.ShapeDtypeStruct((_NP, _D), jnp.float32),
    )(xp, h1s, aggp, dinv1, b1)


def _combine2_body(x1_ref, h2_ref, h3_ref, agg2_ref, agg3_ref, dinv_ref,
                   b2_ref, b3_ref, out_ref):
    a2 = (agg2_ref[0] + agg2_ref[1] + h2_ref[...]) * dinv_ref[0][:, None] \
        + b2_ref[...][None, :]
    a3 = (agg3_ref[0] + agg3_ref[1] + h3_ref[...]) * dinv_ref[1][:, None] \
        + b3_ref[...][None, :]
    t = x1_ref[...] + jnp.maximum(a2 + a3, 0.0)
    nrm = jnp.sqrt(jnp.sum(t * t, axis=-1, keepdims=True))
    out_ref[...] = t / jnp.maximum(nrm, 1e-12)


def _combine2(x1, h2s, h3s, agg2, agg3, dinv, b2, b3):
    return pl.pallas_call(
        _combine2_body,
        grid=(_NB,),
        in_specs=[
            pl.BlockSpec((_BM, _D), lambda i: (i, 0)),
            pl.BlockSpec((_BM, _D), lambda i: (i, 0)),
            pl.BlockSpec((_BM, _D), lambda i: (i, 0)),
            pl.BlockSpec((2, _BM, _D), lambda i: (0, i, 0)),
            pl.BlockSpec((2, _BM, _D), lambda i: (0, i, 0)),
            pl.BlockSpec((2, _BM), lambda i: (0, i)),
            pl.BlockSpec((_D,), lambda i: (0,)),
            pl.BlockSpec((_D,), lambda i: (0,)),
        ],
        out_specs=pl.BlockSpec((_BM, _D), lambda i: (i, 0)),
        out_shape=jax.ShapeDtypeStruct((_NP, _D), jnp.float32),
    )(x1, h2s, h3s, agg2, agg3, dinv, b2, b3)


def kernel(x, edge_index, edge_attr, W1, b1, W2, b2, W3, b3):
    src = edge_index[0]
    dst = edge_index[1]
    xp = jnp.pad(x, ((0, _NP - _N), (0, 0)))
    pad_e = _E2 - _E
    dstp = jnp.pad(dst, (0, pad_e))
    attrp = jnp.pad(edge_attr, (0, pad_e))
    srcp = jnp.concatenate(
        [src, jnp.arange(pad_e, dtype=jnp.int32) % _N])

    s1, s2 = _idx_prep(dstp.reshape(_E2 // 128, 128),
                       attrp.reshape(_E2 // 128, 128))
    s1 = s1.reshape(_E2)
    s2 = s2.reshape(_E2)
    s1_3d = s1.reshape(_NW, _NC, _CH)
    s2_3d = s2.reshape(_NW, _NC, _CH)

    ones_c = jnp.ones((_CH,), jnp.float32)
    zeros_r = jnp.zeros((_RPS,), jnp.float32)
    zeros_b = jnp.zeros((64, _D), jnp.float32)
    spread = jnp.arange(_EW, dtype=jnp.int32)
    padg = spread % _N
    padd = _N + (spread % 128)

    degp = _deg_kernel(s1_3d, s2_3d, ones_c, zeros_r)
    dinv = _dinv(degp.reshape(2, 2, _NP))

    cg1, cd1, cg2, cd2, cnts = _compact_kernel(
        s1.reshape(_NW, _EW), s2.reshape(_NW, _EW),
        srcp.reshape(_NW, _EW), padg, padd)
    cnt1 = jnp.pad(cnts.reshape(_NW, 16)[:, 0], (0, 16))
    cnt2 = jnp.pad(cnts.reshape(_NW, 16)[:, 1], (0, 16))
    cg1 = cg1.reshape(_NSBT, _SB, _CH)
    cd1 = cd1.reshape(_NSBT, _SB, _CH)
    cg2 = cg2.reshape(_NSBT, _SB, _CH)
    cd2 = cd2.reshape(_NSBT, _SB, _CH)

    h1s, h3s = _mm2(xp, xp, W1, W3, dinv)
    agg1 = _agg_kernel(h1s, cg1, cd1, zeros_b, cnt1)
    agg3 = _agg_kernel(h3s, cg2, cd2, zeros_b, cnt2)
    x1 = _combine1(xp, h1s, agg1, dinv[0], b1)

    h2s = _mm_scale(x1, W2, dinv[0])
    agg2 = _agg_kernel(h2s, cg1, cd1, zeros_b, cnt1)
    x2 = _combine2(x1, h2s, h3s, agg2, agg3, dinv, b2, b3)

    return x2[:_N]
